# Initial kernel scaffold; baseline (speedup 1.0000x reference)
import jax
import jax.numpy as jnp
from jax.experimental import pallas as pl


def kernel(x, edge_index, x_batch, W1, b1, W2, b2, Wh, bh):
    raise NotImplementedError("write your pallas kernel here")


# speedup vs baseline: 17.9023x; 17.9023x over previous
import functools

import jax
import jax.numpy as jnp
from jax import lax
from jax.experimental import pallas as pl
from jax.experimental.pallas import tpu as pltpu
from jax.experimental.pallas import tpu_sc as plsc

N = 10000
E = 320000
D = 128
DOUT = 16
G = 8

NW = 32
EPW = E // NW
K = 80
NCH = EPW // K
NSUB = 16
RQ = 624
RL = N - (NSUB - 1) * RQ
BLK = 1000
NBLK = N // BLK


def _copy_slices(s, src_fn, dst_fn):
    @pl.when(s < NSUB - 1)
    def _():
        pltpu.sync_copy(src_fn(s * RQ, RQ), dst_fn(s * RQ, RQ))

    @pl.when(s == NSUB - 1)
    def _():
        pltpu.sync_copy(src_fn((NSUB - 1) * RQ, RL),
                        dst_fn((NSUB - 1) * RQ, RL))


def _deg_body(dstr, zeros128, ones_hbm, out, idx_d, ones_v, accd):
    c = lax.axis_index("c")
    s = lax.axis_index("s")
    wid = s * 2 + c
    _copy_slices(s, lambda o, n: zeros128.at[pl.ds(0, n), :],
                 lambda o, n: accd.at[pl.ds(o, n), :])
    pltpu.sync_copy(ones_hbm, ones_v)
    pltpu.sync_copy(dstr.at[wid], idx_d)
    plsc.subcore_barrier()

    def body(j, carry):
        pltpu.sync_copy(ones_v, accd.at[idx_d.at[j]], add=True)
        return carry

    lax.fori_loop(0, NCH, body, 0)
    plsc.subcore_barrier()
    _copy_slices(s, lambda o, n: accd.at[pl.ds(o, n), :],
                 lambda o, n: out.at[c, pl.ds(o, n), :])


@functools.cache
def _sc_calls():
    mesh = plsc.VectorSubcoreMesh(core_axis_name="c", subcore_axis_name="s",
                                  num_cores=2, num_subcores=NSUB)
    deg_call = pl.kernel(
        _deg_body,
        out_type=jax.ShapeDtypeStruct((2, N, D), jnp.float32),
        mesh=mesh,
        scratch_types=[
            pltpu.VMEM((NCH, K), jnp.int32),
            pltpu.VMEM((K, D), jnp.float32),
            pltpu.VMEM_SHARED((N, D), jnp.float32),
        ],
    )
    agg_call = pl.kernel(
        _agg_body,
        out_type=jax.ShapeDtypeStruct((2, N, D), jnp.float32),
        mesh=mesh,
        scratch_types=[
            pltpu.VMEM((NCH, K), jnp.int32),
            pltpu.VMEM((NCH, K), jnp.int32),
            pltpu.VMEM((K, D), jnp.float32),
            pltpu.VMEM_SHARED((N, D), jnp.float32),
            pltpu.SemaphoreType.DMA,
        ],
    )
    return deg_call, agg_call


def _agg_body(g_hbm, srcr, dstr, zeros128, out, idx_s, idx_d, rows, acc, sem):
    c = lax.axis_index("c")
    s = lax.axis_index("s")
    wid = s * 2 + c
    _copy_slices(s, lambda o, n: zeros128.at[pl.ds(0, n), :],
                 lambda o, n: acc.at[pl.ds(o, n), :])
    pltpu.sync_copy(srcr.at[wid], idx_s)
    pltpu.sync_copy(dstr.at[wid], idx_d)
    plsc.subcore_barrier()

    def body(j, carry):
        pltpu.async_copy(g_hbm.at[idx_s.at[j]], rows, sem).wait()
        pltpu.sync_copy(rows, acc.at[idx_d.at[j]], add=True)
        return carry

    lax.fori_loop(0, NCH, body, 0)
    plsc.subcore_barrier()
    _copy_slices(s, lambda o, n: acc.at[pl.ds(o, n), :],
                 lambda o, n: out.at[c, pl.ds(o, n), :])


def _dinv_from(degp):
    deg = degp[0][:, 0:1] + degp[1][:, 0:1] + 1.0
    return lax.rsqrt(deg)


def _tc_first_body(degp, x, w1, g1):
    dinv = _dinv_from(degp)
    g1[...] = jnp.dot(x[...], w1[...],
                      preferred_element_type=jnp.float32) * dinv


def _tc_mid_body(degp, agg, g1, b1, w2, g2):
    dinv = _dinv_from(degp)
    h = jnp.maximum((agg[0] + agg[1] + g1[...]) * dinv + b1[...], 0.0)
    g2[...] = jnp.dot(h, w2[...], preferred_element_type=jnp.float32) * dinv


def _tc_last_body(degp, agg, g2, b2, xb, wh, bh, out, pooled):
    i = pl.program_id(0)
    dinv = _dinv_from(degp)
    h = jnp.maximum((agg[0] + agg[1] + g2[...]) * dinv + b2[...], 0.0)
    gi = lax.broadcasted_iota(jnp.int32, (G, BLK), 0)
    m = (gi == xb[0]).astype(jnp.float32)
    pp = jnp.dot(m, h, preferred_element_type=jnp.float32)

    @pl.when(i == 0)
    def _():
        pooled[...] = pp

    @pl.when(i > 0)
    def _():
        pooled[...] += pp

    @pl.when(i == NBLK - 1)
    def _():
        o = jnp.dot(pooled[...], wh[...],
                    preferred_element_type=jnp.float32) + bh[...]
        z = o - jnp.max(o, axis=1, keepdims=True)
        lse = jnp.log(jnp.sum(jnp.exp(z), axis=1, keepdims=True))
        out[...] = z - lse


def _degp_spec():
    return pl.BlockSpec((2, BLK, D), lambda i: (0, i, 0))


def _rows_spec():
    return pl.BlockSpec((BLK, D), lambda i: (i, 0))


def _agg_spec():
    return pl.BlockSpec((2, BLK, D), lambda i: (0, i, 0))


def _full_spec(shape):
    nd = len(shape)
    return pl.BlockSpec(shape, lambda i: (0,) * nd)


_tc_first = pl.pallas_call(
    _tc_first_body,
    grid=(NBLK,),
    in_specs=[_degp_spec(), _rows_spec(), _full_spec((D, D))],
    out_specs=_rows_spec(),
    out_shape=jax.ShapeDtypeStruct((N, D), jnp.float32),
)

_tc_mid = pl.pallas_call(
    _tc_mid_body,
    grid=(NBLK,),
    in_specs=[_degp_spec(), _agg_spec(), _rows_spec(),
              _full_spec((1, D)), _full_spec((D, D))],
    out_specs=_rows_spec(),
    out_shape=jax.ShapeDtypeStruct((N, D), jnp.float32),
)

_tc_last = pl.pallas_call(
    _tc_last_body,
    grid=(NBLK,),
    in_specs=[_degp_spec(), _agg_spec(), _rows_spec(),
              _full_spec((1, D)),
              pl.BlockSpec((1, 1, BLK), lambda i: (i, 0, 0)),
              _full_spec((D, DOUT)), _full_spec((1, DOUT))],
    out_specs=_full_spec((G, DOUT)),
    out_shape=jax.ShapeDtypeStruct((G, DOUT), jnp.float32),
    scratch_shapes=[pltpu.VMEM((G, D), jnp.float32)],
)


def kernel(x, edge_index, x_batch, W1, b1, W2, b2, Wh, bh):
    srcr = edge_index[0].reshape(NW, NCH, K)
    dstr = edge_index[1].reshape(NW, NCH, K)
    zeros128 = jnp.zeros((RL, D), jnp.float32)
    ones128 = jnp.ones((K, D), jnp.float32)
    xb = x_batch.reshape(NBLK, 1, BLK)
    b1r = b1.reshape(1, D)
    b2r = b2.reshape(1, D)
    bhr = bh.reshape(1, DOUT)

    deg_call, agg_call = _sc_calls()
    degp = deg_call(dstr, zeros128, ones128)
    g1 = _tc_first(degp, x, W1)
    agg1 = agg_call(g1, srcr, dstr, zeros128)
    g2 = _tc_mid(degp, agg1, g1, b1r, W2)
    agg2 = agg_call(g2, srcr, dstr, zeros128)
    return _tc_last(degp, agg2, g2, b2r, xb, Wh, bhr)


# speedup vs baseline: 21.9106x; 1.2239x over previous
import functools

import jax
import jax.numpy as jnp
from jax import lax
from jax.experimental import pallas as pl
from jax.experimental.pallas import tpu as pltpu
from jax.experimental.pallas import tpu_sc as plsc

N = 10000
E = 320000
D = 128
DOUT = 16
G = 8

NW = 32
EPW = E // NW
K = 80
NCH = EPW // K
NSUB = 16
RQ = 624
RL = N - (NSUB - 1) * RQ
BLK = 1000
NBLK = N // BLK


def _copy_slices(s, src_fn, dst_fn):
    @pl.when(s < NSUB - 1)
    def _():
        pltpu.sync_copy(src_fn(s * RQ, RQ), dst_fn(s * RQ, RQ))

    @pl.when(s == NSUB - 1)
    def _():
        pltpu.sync_copy(src_fn((NSUB - 1) * RQ, RL),
                        dst_fn((NSUB - 1) * RQ, RL))


def _deg_body(dstr, zeros128, ones_hbm, out, idx_d, ones_v, accd, ssem):
    c = lax.axis_index("c")
    s = lax.axis_index("s")
    wid = s * 2 + c
    _copy_slices(s, lambda o, n: zeros128.at[pl.ds(0, n), :],
                 lambda o, n: accd.at[pl.ds(o, n), :])
    pltpu.sync_copy(ones_hbm, ones_v)
    pltpu.sync_copy(dstr.at[wid], idx_d)
    plsc.subcore_barrier()

    def body(t, carry):
        for u in range(5):
            pltpu.async_copy(ones_v, accd.at[idx_d.at[t * 5 + u]], ssem,
                             add=True)
        for u in range(5):
            pltpu.make_async_copy(ones_v, accd.at[idx_d.at[t * 5 + u]],
                                  ssem).wait()
        return carry

    lax.fori_loop(0, NCH // 5, body, 0)
    plsc.subcore_barrier()
    _copy_slices(s, lambda o, n: accd.at[pl.ds(o, n), :],
                 lambda o, n: out.at[c, pl.ds(o, n), :])


@functools.cache
def _sc_calls():
    mesh = plsc.VectorSubcoreMesh(core_axis_name="c", subcore_axis_name="s",
                                  num_cores=2, num_subcores=NSUB)
    deg_call = pl.kernel(
        _deg_body,
        out_type=jax.ShapeDtypeStruct((2, N, D), jnp.float32),
        mesh=mesh,
        scratch_types=[
            pltpu.VMEM((NCH, K), jnp.int32),
            pltpu.VMEM((K, D), jnp.float32),
            pltpu.VMEM_SHARED((N, D), jnp.float32),
            pltpu.SemaphoreType.DMA,
        ],
    )
    agg_call = pl.kernel(
        _agg_body,
        out_type=jax.ShapeDtypeStruct((2, N, D), jnp.float32),
        mesh=mesh,
        scratch_types=[
            pltpu.VMEM((EPW,), jnp.int32),
            pltpu.VMEM((NCH, K), jnp.int32),
            pltpu.VMEM((K, D), jnp.float32),
            pltpu.VMEM((K, D), jnp.float32),
            pltpu.VMEM_SHARED((N, D), jnp.float32),
            pltpu.SemaphoreType.DMA,
            pltpu.SemaphoreType.DMA,
        ],
    )
    return deg_call, agg_call


def _sidx(idx_s, j):
    return idx_s.at[pl.ds(j * K, K)]


def _agg_body(g_hbm, srcf, dstr, zeros128, out,
              idx_s, idx_d, rows0, rows1, acc, gsem0, gsem1):
    c = lax.axis_index("c")
    s = lax.axis_index("s")
    wid = s * 2 + c
    _copy_slices(s, lambda o, n: zeros128.at[pl.ds(0, n), :],
                 lambda o, n: acc.at[pl.ds(o, n), :])
    pltpu.sync_copy(srcf.at[pl.ds(wid * EPW, EPW)], idx_s)
    pltpu.sync_copy(dstr.at[wid], idx_d)
    plsc.subcore_barrier()

    pltpu.async_copy(g_hbm.at[_sidx(idx_s, 0)], rows0, gsem0).wait()
    pltpu.async_copy(g_hbm.at[_sidx(idx_s, 1)], rows1, gsem1)
    pltpu.sync_copy(rows0, acc.at[idx_d.at[0]], add=True)

    def body(jj, carry):
        a = 2 * jj + 1
        b = a + 1
        pltpu.make_async_copy(g_hbm.at[_sidx(idx_s, a)], rows1, gsem1).wait()
        pltpu.async_copy(g_hbm.at[_sidx(idx_s, b)], rows0, gsem0)
        pltpu.sync_copy(rows1, acc.at[idx_d.at[a]], add=True)
        pltpu.make_async_copy(g_hbm.at[_sidx(idx_s, b)], rows0, gsem0).wait()
        pltpu.async_copy(g_hbm.at[_sidx(idx_s, b + 1)], rows1, gsem1)
        pltpu.sync_copy(rows0, acc.at[idx_d.at[b]], add=True)
        return carry

    lax.fori_loop(0, (NCH - 3) // 2, body, 0)
    pltpu.make_async_copy(g_hbm.at[_sidx(idx_s, NCH - 2)], rows1, gsem1).wait()
    pltpu.async_copy(g_hbm.at[_sidx(idx_s, NCH - 1)], rows0, gsem0)
    pltpu.sync_copy(rows1, acc.at[idx_d.at[NCH - 2]], add=True)
    pltpu.make_async_copy(g_hbm.at[_sidx(idx_s, NCH - 1)], rows0, gsem0).wait()
    pltpu.sync_copy(rows0, acc.at[idx_d.at[NCH - 1]], add=True)
    plsc.subcore_barrier()
    _copy_slices(s, lambda o, n: acc.at[pl.ds(o, n), :],
                 lambda o, n: out.at[c, pl.ds(o, n), :])


def _dinv_from(degp):
    deg = degp[0][:, 0:1] + degp[1][:, 0:1] + 1.0
    return lax.rsqrt(deg)


def _tc_first_body(degp, x, w1, g1):
    dinv = _dinv_from(degp)
    g1[...] = jnp.dot(x[...], w1[...],
                      preferred_element_type=jnp.float32) * dinv


def _tc_mid_body(degp, agg, g1, b1, w2, g2):
    dinv = _dinv_from(degp)
    h = jnp.maximum((agg[0] + agg[1] + g1[...]) * dinv + b1[...], 0.0)
    g2[...] = jnp.dot(h, w2[...], preferred_element_type=jnp.float32) * dinv


def _tc_last_body(degp, agg, g2, b2, xb, wh, bh, out, pooled):
    i = pl.program_id(0)
    dinv = _dinv_from(degp)
    h = jnp.maximum((agg[0] + agg[1] + g2[...]) * dinv + b2[...], 0.0)
    gi = lax.broadcasted_iota(jnp.int32, (G, BLK), 0)
    m = (gi == xb[0]).astype(jnp.float32)
    pp = jnp.dot(m, h, preferred_element_type=jnp.float32)

    @pl.when(i == 0)
    def _():
        pooled[...] = pp

    @pl.when(i > 0)
    def _():
        pooled[...] += pp

    @pl.when(i == NBLK - 1)
    def _():
        o = jnp.dot(pooled[...], wh[...],
                    preferred_element_type=jnp.float32) + bh[...]
        z = o - jnp.max(o, axis=1, keepdims=True)
        lse = jnp.log(jnp.sum(jnp.exp(z), axis=1, keepdims=True))
        out[...] = z - lse


def _degp_spec():
    return pl.BlockSpec((2, BLK, D), lambda i: (0, i, 0))


def _rows_spec():
    return pl.BlockSpec((BLK, D), lambda i: (i, 0))


def _agg_spec():
    return pl.BlockSpec((2, BLK, D), lambda i: (0, i, 0))


def _full_spec(shape):
    nd = len(shape)
    return pl.BlockSpec(shape, lambda i: (0,) * nd)


_tc_first = pl.pallas_call(
    _tc_first_body,
    grid=(NBLK,),
    in_specs=[_degp_spec(), _rows_spec(), _full_spec((D, D))],
    out_specs=_rows_spec(),
    out_shape=jax.ShapeDtypeStruct((N, D), jnp.float32),
)

_tc_mid = pl.pallas_call(
    _tc_mid_body,
    grid=(NBLK,),
    in_specs=[_degp_spec(), _agg_spec(), _rows_spec(),
              _full_spec((1, D)), _full_spec((D, D))],
    out_specs=_rows_spec(),
    out_shape=jax.ShapeDtypeStruct((N, D), jnp.float32),
)

_tc_last = pl.pallas_call(
    _tc_last_body,
    grid=(NBLK,),
    in_specs=[_degp_spec(), _agg_spec(), _rows_spec(),
              _full_spec((1, D)),
              pl.BlockSpec((1, 1, BLK), lambda i: (i, 0, 0)),
              _full_spec((D, DOUT)), _full_spec((1, DOUT))],
    out_specs=_full_spec((G, DOUT)),
    out_shape=jax.ShapeDtypeStruct((G, DOUT), jnp.float32),
    scratch_shapes=[pltpu.VMEM((G, D), jnp.float32)],
)


def kernel(x, edge_index, x_batch, W1, b1, W2, b2, Wh, bh):
    srcf = edge_index[0]
    dstr = edge_index[1].reshape(NW, NCH, K)
    zeros128 = jnp.zeros((RL, D), jnp.float32)
    ones128 = jnp.ones((K, D), jnp.float32)
    xb = x_batch.reshape(NBLK, 1, BLK)
    b1r = b1.reshape(1, D)
    b2r = b2.reshape(1, D)
    bhr = bh.reshape(1, DOUT)

    deg_call, agg_call = _sc_calls()
    degp = deg_call(dstr, zeros128, ones128)
    g1 = _tc_first(degp, x, W1)
    agg1 = agg_call(g1, srcf, dstr, zeros128)
    g2 = _tc_mid(degp, agg1, g1, b1r, W2)
    agg2 = agg_call(g2, srcf, dstr, zeros128)
    return _tc_last(degp, agg2, g2, b2r, xb, Wh, bhr)


# speedup vs baseline: 24.5319x; 1.1196x over previous
import functools

import jax
import jax.numpy as jnp
from jax import lax
from jax.experimental import pallas as pl
from jax.experimental.pallas import tpu as pltpu
from jax.experimental.pallas import tpu_sc as plsc

N = 10000
E = 320000
D = 128
DOUT = 16
G = 8

NW = 32
EPW = E // NW
K = 80
NCH = EPW // K
NSUB = 16
RQ = 624
RL = N - (NSUB - 1) * RQ
BLK = 1000
NBLK = N // BLK
NPAD = 10240
HR = NPAD // 128


def _copy_slices(s, src_fn, dst_fn):
    @pl.when(s < NSUB - 1)
    def _():
        pltpu.sync_copy(src_fn(s * RQ, RQ), dst_fn(s * RQ, RQ))

    @pl.when(s == NSUB - 1)
    def _():
        pltpu.sync_copy(src_fn((NSUB - 1) * RQ, RL),
                        dst_fn((NSUB - 1) * RQ, RL))


def _deg_body(dstf, zeros128, out, idx_v, hist):
    c = lax.axis_index("c")
    s = lax.axis_index("s")
    wid = s * 2 + c
    pltpu.sync_copy(zeros128.at[pl.ds(0, HR), :], hist)
    pltpu.sync_copy(dstf.at[pl.ds(wid * EPW, EPW)], idx_v)
    ones = jnp.ones((16,), jnp.float32)

    def body(k, carry):
        iv = idx_v[pl.ds(k * 16, 16)]
        row = lax.shift_right_logical(iv, 7)
        col = lax.bitwise_and(iv, 127)
        plsc.addupdate_scatter(hist, [row, col], ones)
        return carry

    lax.fori_loop(0, EPW // 16, body, 0)
    pltpu.sync_copy(hist, out.at[wid])


@functools.cache
def _sc_calls():
    mesh = plsc.VectorSubcoreMesh(core_axis_name="c", subcore_axis_name="s",
                                  num_cores=2, num_subcores=NSUB)
    deg_call = pl.kernel(
        _deg_body,
        out_type=jax.ShapeDtypeStruct((NW, HR, 128), jnp.float32),
        compiler_params=pltpu.CompilerParams(needs_layout_passes=False),
        mesh=mesh,
        scratch_types=[
            pltpu.VMEM((EPW,), jnp.int32),
            pltpu.VMEM((HR, 128), jnp.float32),
        ],
    )
    agg_call = pl.kernel(
        _agg_body,
        out_type=jax.ShapeDtypeStruct((2, N, D), jnp.float32),
        mesh=mesh,
        scratch_types=[
            pltpu.VMEM((EPW,), jnp.int32),
            pltpu.VMEM((NCH, K), jnp.int32),
            pltpu.VMEM((K, D), jnp.float32),
            pltpu.VMEM((K, D), jnp.float32),
            pltpu.VMEM_SHARED((N, D), jnp.float32),
            pltpu.SemaphoreType.DMA,
            pltpu.SemaphoreType.DMA,
        ],
    )
    return deg_call, agg_call


def _sidx(idx_s, j):
    return idx_s.at[pl.ds(j * K, K)]


def _agg_body(g_hbm, srcf, dstr, zeros128, out,
              idx_s, idx_d, rows0, rows1, acc, gsem0, gsem1):
    c = lax.axis_index("c")
    s = lax.axis_index("s")
    wid = s * 2 + c
    _copy_slices(s, lambda o, n: zeros128.at[pl.ds(0, n), :],
                 lambda o, n: acc.at[pl.ds(o, n), :])
    pltpu.sync_copy(srcf.at[pl.ds(wid * EPW, EPW)], idx_s)
    pltpu.sync_copy(dstr.at[wid], idx_d)
    plsc.subcore_barrier()

    pltpu.async_copy(g_hbm.at[_sidx(idx_s, 0)], rows0, gsem0).wait()
    pltpu.async_copy(g_hbm.at[_sidx(idx_s, 1)], rows1, gsem1)
    pltpu.sync_copy(rows0, acc.at[idx_d.at[0]], add=True)

    def body(jj, carry):
        a = 2 * jj + 1
        b = a + 1
        pltpu.make_async_copy(g_hbm.at[_sidx(idx_s, a)], rows1, gsem1).wait()
        pltpu.async_copy(g_hbm.at[_sidx(idx_s, b)], rows0, gsem0)
        pltpu.sync_copy(rows1, acc.at[idx_d.at[a]], add=True)
        pltpu.make_async_copy(g_hbm.at[_sidx(idx_s, b)], rows0, gsem0).wait()
        pltpu.async_copy(g_hbm.at[_sidx(idx_s, b + 1)], rows1, gsem1)
        pltpu.sync_copy(rows0, acc.at[idx_d.at[b]], add=True)
        return carry

    lax.fori_loop(0, (NCH - 3) // 2, body, 0)
    pltpu.make_async_copy(g_hbm.at[_sidx(idx_s, NCH - 2)], rows1, gsem1).wait()
    pltpu.async_copy(g_hbm.at[_sidx(idx_s, NCH - 1)], rows0, gsem0)
    pltpu.sync_copy(rows1, acc.at[idx_d.at[NCH - 2]], add=True)
    pltpu.make_async_copy(g_hbm.at[_sidx(idx_s, NCH - 1)], rows0, gsem0).wait()
    pltpu.sync_copy(rows0, acc.at[idx_d.at[NCH - 1]], add=True)
    plsc.subcore_barrier()
    _copy_slices(s, lambda o, n: acc.at[pl.ds(o, n), :],
                 lambda o, n: out.at[c, pl.ds(o, n), :])


def _tc_deg_body(d32, dinv_out):
    deg = jnp.sum(d32[...], axis=0, keepdims=True) + 1.0
    dinv_out[...] = jnp.transpose(lax.rsqrt(deg))


def _tc_first_body(dv, x, w1, g1):
    dinv = dv[...]
    g1[...] = jnp.dot(x[...], w1[...],
                      preferred_element_type=jnp.float32) * dinv


def _tc_mid_body(dv, agg, g1, b1, w2, g2):
    dinv = dv[...]
    h = jnp.maximum((agg[0] + agg[1] + g1[...]) * dinv + b1[...], 0.0)
    g2[...] = jnp.dot(h, w2[...], preferred_element_type=jnp.float32) * dinv


def _tc_last_body(dv, agg, g2, b2, xb, wh, bh, out, pooled):
    i = pl.program_id(0)
    dinv = dv[...]
    h = jnp.maximum((agg[0] + agg[1] + g2[...]) * dinv + b2[...], 0.0)
    gi = lax.broadcasted_iota(jnp.int32, (G, BLK), 0)
    m = (gi == xb[0]).astype(jnp.float32)
    pp = jnp.dot(m, h, preferred_element_type=jnp.float32)

    @pl.when(i == 0)
    def _():
        pooled[...] = pp

    @pl.when(i > 0)
    def _():
        pooled[...] += pp

    @pl.when(i == NBLK - 1)
    def _():
        o = jnp.dot(pooled[...], wh[...],
                    preferred_element_type=jnp.float32) + bh[...]
        z = o - jnp.max(o, axis=1, keepdims=True)
        lse = jnp.log(jnp.sum(jnp.exp(z), axis=1, keepdims=True))
        out[...] = z - lse


def _dinv_spec():
    return pl.BlockSpec((BLK, 1), lambda i: (i, 0))


def _rows_spec():
    return pl.BlockSpec((BLK, D), lambda i: (i, 0))


def _agg_spec():
    return pl.BlockSpec((2, BLK, D), lambda i: (0, i, 0))


def _full_spec(shape):
    nd = len(shape)
    return pl.BlockSpec(shape, lambda i: (0,) * nd)


_tc_deg = pl.pallas_call(
    _tc_deg_body,
    in_specs=[pl.BlockSpec((NW, NPAD), lambda: (0, 0))],
    out_specs=pl.BlockSpec((NPAD, 1), lambda: (0, 0)),
    out_shape=jax.ShapeDtypeStruct((NPAD, 1), jnp.float32),
)

_tc_first = pl.pallas_call(
    _tc_first_body,
    grid=(NBLK,),
    in_specs=[_dinv_spec(), _rows_spec(), _full_spec((D, D))],
    out_specs=_rows_spec(),
    out_shape=jax.ShapeDtypeStruct((N, D), jnp.float32),
)

_tc_mid = pl.pallas_call(
    _tc_mid_body,
    grid=(NBLK,),
    in_specs=[_dinv_spec(), _agg_spec(), _rows_spec(),
              _full_spec((1, D)), _full_spec((D, D))],
    out_specs=_rows_spec(),
    out_shape=jax.ShapeDtypeStruct((N, D), jnp.float32),
)

_tc_last = pl.pallas_call(
    _tc_last_body,
    grid=(NBLK,),
    in_specs=[_dinv_spec(), _agg_spec(), _rows_spec(),
              _full_spec((1, D)),
              pl.BlockSpec((1, 1, BLK), lambda i: (i, 0, 0)),
              _full_spec((D, DOUT)), _full_spec((1, DOUT))],
    out_specs=_full_spec((G, DOUT)),
    out_shape=jax.ShapeDtypeStruct((G, DOUT), jnp.float32),
    scratch_shapes=[pltpu.VMEM((G, D), jnp.float32)],
)


def kernel(x, edge_index, x_batch, W1, b1, W2, b2, Wh, bh):
    srcf = edge_index[0]
    dstf = edge_index[1]
    dstr = dstf.reshape(NW, NCH, K)
    zeros128 = jnp.zeros((RL, D), jnp.float32)
    xb = x_batch.reshape(NBLK, 1, BLK)
    b1r = b1.reshape(1, D)
    b2r = b2.reshape(1, D)
    bhr = bh.reshape(1, DOUT)

    deg_call, agg_call = _sc_calls()
    degp32 = deg_call(dstf, zeros128)
    dinv_pad = _tc_deg(degp32.reshape(NW, NPAD))
    dinv = dinv_pad[:N]
    g1 = _tc_first(dinv, x, W1)
    agg1 = agg_call(g1, srcf, dstr, zeros128)
    g2 = _tc_mid(dinv, agg1, g1, b1r, W2)
    agg2 = agg_call(g2, srcf, dstr, zeros128)
    return _tc_last(dinv, agg2, g2, b2r, xb, Wh, bhr)


# speedup vs baseline: 24.9402x; 1.0166x over previous
import functools

import jax
import jax.numpy as jnp
from jax import lax
from jax.experimental import pallas as pl
from jax.experimental.pallas import tpu as pltpu
from jax.experimental.pallas import tpu_sc as plsc

N = 10000
E = 320000
D = 128
DOUT = 16
G = 8

NW = 32
EPW = E // NW
K = 80
NCH = EPW // K
NSUB = 16
RQ = 624
RL = N - (NSUB - 1) * RQ
BLK = 1000
NBLK = N // BLK
NPAD = 10240
HR = NPAD // 128


def _copy_slices(s, src_fn, dst_fn):
    @pl.when(s < NSUB - 1)
    def _():
        pltpu.sync_copy(src_fn(s * RQ, RQ), dst_fn(s * RQ, RQ))

    @pl.when(s == NSUB - 1)
    def _():
        pltpu.sync_copy(src_fn((NSUB - 1) * RQ, RL),
                        dst_fn((NSUB - 1) * RQ, RL))


def _deg_body(dstf, zeros128, out, idx_v, hist):
    c = lax.axis_index("c")
    s = lax.axis_index("s")
    wid = s * 2 + c
    pltpu.sync_copy(zeros128.at[pl.ds(0, HR), :], hist)
    pltpu.sync_copy(dstf.at[pl.ds(wid * EPW, EPW)], idx_v)
    ones = jnp.ones((16,), jnp.float32)

    def body(k, carry):
        iv = idx_v[pl.ds(k * 16, 16)]
        row = lax.shift_right_logical(iv, 7)
        col = lax.bitwise_and(iv, 127)
        plsc.addupdate_scatter(hist, [row, col], ones)
        return carry

    lax.fori_loop(0, EPW // 16, body, 0)
    pltpu.sync_copy(hist, out.at[wid])


@functools.cache
def _sc_calls():
    mesh = plsc.VectorSubcoreMesh(core_axis_name="c", subcore_axis_name="s",
                                  num_cores=2, num_subcores=NSUB)
    deg_call = pl.kernel(
        _deg_body,
        out_type=jax.ShapeDtypeStruct((NW, HR, 128), jnp.float32),
        compiler_params=pltpu.CompilerParams(needs_layout_passes=False),
        mesh=mesh,
        scratch_types=[
            pltpu.VMEM((EPW,), jnp.int32),
            pltpu.VMEM((HR, 128), jnp.float32),
        ],
    )
    agg_call = pl.kernel(
        _agg_body,
        out_type=jax.ShapeDtypeStruct((2, N, D), jnp.float32),
        mesh=mesh,
        scratch_types=[
            pltpu.VMEM((EPW,), jnp.int32),
            pltpu.VMEM((NCH, K), jnp.int32),
            pltpu.VMEM((K, D), jnp.float32),
            pltpu.VMEM((K, D), jnp.float32),
            pltpu.VMEM_SHARED((N, D), jnp.float32),
            pltpu.SemaphoreType.DMA,
            pltpu.SemaphoreType.DMA,
        ],
    )
    return deg_call, agg_call


def _sidx(idx_s, j):
    return idx_s.at[pl.ds(j * K, K)]


def _agg_body(g_hbm, srcf, dstr, zeros128, out,
              idx_s, idx_d, rows0, rows1, acc, gsem0, gsem1):
    c = lax.axis_index("c")
    s = lax.axis_index("s")
    wid = s * 2 + c
    _copy_slices(s, lambda o, n: zeros128.at[pl.ds(0, n), :],
                 lambda o, n: acc.at[pl.ds(o, n), :])
    pltpu.sync_copy(srcf.at[pl.ds(wid * EPW, EPW)], idx_s)
    pltpu.sync_copy(dstr.at[wid], idx_d)
    plsc.subcore_barrier()

    pltpu.async_copy(g_hbm.at[_sidx(idx_s, 0)], rows0, gsem0).wait()
    pltpu.async_copy(g_hbm.at[_sidx(idx_s, 1)], rows1, gsem1)
    pltpu.sync_copy(rows0, acc.at[idx_d.at[0]], add=True)

    def body(jj, carry):
        a = 2 * jj + 1
        b = a + 1
        pltpu.make_async_copy(g_hbm.at[_sidx(idx_s, a)], rows1, gsem1).wait()
        pltpu.async_copy(g_hbm.at[_sidx(idx_s, b)], rows0, gsem0)
        pltpu.sync_copy(rows1, acc.at[idx_d.at[a]], add=True)
        pltpu.make_async_copy(g_hbm.at[_sidx(idx_s, b)], rows0, gsem0).wait()
        pltpu.async_copy(g_hbm.at[_sidx(idx_s, b + 1)], rows1, gsem1)
        pltpu.sync_copy(rows0, acc.at[idx_d.at[b]], add=True)
        return carry

    lax.fori_loop(0, (NCH - 3) // 2, body, 0)
    pltpu.make_async_copy(g_hbm.at[_sidx(idx_s, NCH - 2)], rows1, gsem1).wait()
    pltpu.async_copy(g_hbm.at[_sidx(idx_s, NCH - 1)], rows0, gsem0)
    pltpu.sync_copy(rows1, acc.at[idx_d.at[NCH - 2]], add=True)
    pltpu.make_async_copy(g_hbm.at[_sidx(idx_s, NCH - 1)], rows0, gsem0).wait()
    pltpu.sync_copy(rows0, acc.at[idx_d.at[NCH - 1]], add=True)
    plsc.subcore_barrier()
    _copy_slices(s, lambda o, n: acc.at[pl.ds(o, n), :],
                 lambda o, n: out.at[c, pl.ds(o, n), :])


def _dinv_from(dg):
    deg = jnp.sum(dg[...], axis=1, keepdims=True) + 1.0
    return lax.rsqrt(deg)


def _tc_first_body(dg, x, w1, g1):
    dinv = _dinv_from(dg)
    g1[...] = jnp.dot(x[...], w1[...],
                      preferred_element_type=jnp.float32) * dinv


def _tc_mid_body(dg, agg, g1, b1, w2, g2):
    dinv = _dinv_from(dg)
    h = jnp.maximum((agg[0] + agg[1] + g1[...]) * dinv + b1[...], 0.0)
    g2[...] = jnp.dot(h, w2[...], preferred_element_type=jnp.float32) * dinv


def _tc_last_body(dg, agg, g2, b2, xb, wh, bh, out, pooled):
    i = pl.program_id(0)
    dinv = _dinv_from(dg)
    h = jnp.maximum((agg[0] + agg[1] + g2[...]) * dinv + b2[...], 0.0)
    gi = lax.broadcasted_iota(jnp.int32, (G, BLK), 0)
    m = (gi == xb[0]).astype(jnp.float32)
    pp = jnp.dot(m, h, preferred_element_type=jnp.float32)

    @pl.when(i == 0)
    def _():
        pooled[...] = pp

    @pl.when(i > 0)
    def _():
        pooled[...] += pp

    @pl.when(i == NBLK - 1)
    def _():
        o = jnp.dot(pooled[...], wh[...],
                    preferred_element_type=jnp.float32) + bh[...]
        z = o - jnp.max(o, axis=1, keepdims=True)
        lse = jnp.log(jnp.sum(jnp.exp(z), axis=1, keepdims=True))
        out[...] = z - lse


def _dinv_spec():
    return pl.BlockSpec((BLK, NW), lambda i: (i, 0))


def _rows_spec():
    return pl.BlockSpec((BLK, D), lambda i: (i, 0))


def _agg_spec():
    return pl.BlockSpec((2, BLK, D), lambda i: (0, i, 0))


def _full_spec(shape):
    nd = len(shape)
    return pl.BlockSpec(shape, lambda i: (0,) * nd)


_tc_first = pl.pallas_call(
    _tc_first_body,
    grid=(NBLK,),
    in_specs=[_dinv_spec(), _rows_spec(), _full_spec((D, D))],
    out_specs=_rows_spec(),
    out_shape=jax.ShapeDtypeStruct((N, D), jnp.float32),
)

_tc_mid = pl.pallas_call(
    _tc_mid_body,
    grid=(NBLK,),
    in_specs=[_dinv_spec(), _agg_spec(), _rows_spec(),
              _full_spec((1, D)), _full_spec((D, D))],
    out_specs=_rows_spec(),
    out_shape=jax.ShapeDtypeStruct((N, D), jnp.float32),
)

_tc_last = pl.pallas_call(
    _tc_last_body,
    grid=(NBLK,),
    in_specs=[_dinv_spec(), _agg_spec(), _rows_spec(),
              _full_spec((1, D)),
              pl.BlockSpec((1, 1, BLK), lambda i: (i, 0, 0)),
              _full_spec((D, DOUT)), _full_spec((1, DOUT))],
    out_specs=_full_spec((G, DOUT)),
    out_shape=jax.ShapeDtypeStruct((G, DOUT), jnp.float32),
    scratch_shapes=[pltpu.VMEM((G, D), jnp.float32)],
)


def kernel(x, edge_index, x_batch, W1, b1, W2, b2, Wh, bh):
    srcf = edge_index[0]
    dstf = edge_index[1]
    dstr = dstf.reshape(NW, NCH, K)
    zeros128 = jnp.zeros((RL, D), jnp.float32)
    xb = x_batch.reshape(NBLK, 1, BLK)
    b1r = b1.reshape(1, D)
    b2r = b2.reshape(1, D)
    bhr = bh.reshape(1, DOUT)

    deg_call, agg_call = _sc_calls()
    degp32 = deg_call(dstf, zeros128)
    dg = degp32.reshape(NW, NPAD).T
    g1 = _tc_first(dg, x, W1)
    agg1 = agg_call(g1, srcf, dstr, zeros128)
    g2 = _tc_mid(dg, agg1, g1, b1r, W2)
    agg2 = agg_call(g2, srcf, dstr, zeros128)
    return _tc_last(dg, agg2, g2, b2r, xb, Wh, bhr)
